# Initial kernel scaffold; baseline (speedup 1.0000x reference)
import jax
import jax.numpy as jnp
from jax.experimental import pallas as pl


def kernel(x, W1, b1, ln_g, ln_b, W2, b2, neuron_keys, input_patterns, process_weights, process_outputs, k_input, k_process):
    raise NotImplementedError("write your pallas kernel here")


# speedup vs baseline: 3.2617x; 3.2617x over previous
import functools

import jax
import jax.numpy as jnp
from jax import lax
from jax.experimental import pallas as pl
from jax.experimental.pallas import tpu as pltpu
from jax.experimental.pallas import tpu_sc as plsc

_B, _S, _D = 2, 2048, 768
_N_IN, _N_PROC, _D_R = 4096, 2048, 256
_K_IN, _K_PROC = 2048, 1024
_TEMP = 0.5


def _gelu(v):
    return 0.5 * v * (1.0 + lax.erf(v * (2.0 ** -0.5)))


def _router_body(x_ref, w1_ref, b1_ref, g_ref, bln_ref, w2_ref, b2_ref,
                 nk_ref, logits_ref):
    gc = jnp.max(x_ref[...], axis=1)
    h = lax.dot_general(gc, w1_ref[...], (((1,), (1,)), ((), ())),
                        preferred_element_type=jnp.float32) + b1_ref[...]
    h = _gelu(h)
    mu = jnp.mean(h, axis=-1, keepdims=True)
    var = jnp.mean((h - mu) ** 2, axis=-1, keepdims=True)
    h = (h - mu) / jnp.sqrt(var + 1e-5) * g_ref[...] + bln_ref[...]
    q = lax.dot_general(h, w2_ref[...], (((1,), (1,)), ((), ())),
                        preferred_element_type=jnp.float32) + b2_ref[...]
    logits_ref[...] = lax.dot_general(
        q, nk_ref[...], (((1,), (1,)), ((), ())),
        preferred_element_type=jnp.float32) * (_D_R ** -0.5)


def _router(x, w1, b1, g, bln, w2, b2, nk):
    return pl.pallas_call(
        _router_body,
        out_shape=jax.ShapeDtypeStruct((_B, _N_IN), jnp.float32),
    )(x, w1, b1, g, bln, w2, b2, nk)


def _sc_gather(table, idx_flat, rows_total, d, chunk):
    info = plsc.get_sparse_core_info()
    nc, ns = info.num_cores, info.num_subcores
    nw = nc * ns
    per_w = rows_total // nw
    mesh = plsc.VectorSubcoreMesh(core_axis_name="c", subcore_axis_name="s")

    @functools.partial(
        pl.kernel, mesh=mesh,
        out_type=jax.ShapeDtypeStruct((rows_total, d), jnp.float32),
        scratch_types=[
            pltpu.VMEM((chunk,), jnp.int32),
            pltpu.VMEM((chunk, d), jnp.float32),
            pltpu.SemaphoreType.DMA,
        ],
    )
    def k(table_hbm, idx_hbm, out_hbm, idx_v, rows_v, sem):
        wid = lax.axis_index("s") * nc + lax.axis_index("c")
        base = wid * per_w
        for c in range(per_w // chunk):
            off = base + c * chunk
            pltpu.sync_copy(idx_hbm.at[pl.ds(off, chunk)], idx_v)
            pltpu.async_copy(table_hbm.at[idx_v], rows_v, sem).wait()
            pltpu.sync_copy(rows_v, out_hbm.at[pl.ds(off, chunk)])

    return k(table, idx_flat)


def _selin_body(x_ref, ip_ref, out_ref):
    out_ref[0] = _gelu(lax.dot_general(
        x_ref[0], ip_ref[0], (((1,), (1,)), ((), ())),
        preferred_element_type=jnp.float32))


def _selin(x, ipsel):
    bs = 512
    return pl.pallas_call(
        _selin_body,
        grid=(_B, _S // bs),
        in_specs=[
            pl.BlockSpec((1, bs, _D), lambda b, s: (b, s, 0)),
            pl.BlockSpec((1, _K_IN, _D), lambda b, s: (b, 0, 0)),
        ],
        out_specs=pl.BlockSpec((1, bs, _K_IN), lambda b, s: (b, s, 0)),
        out_shape=jax.ShapeDtypeStruct((_B, _S, _K_IN), jnp.float32),
    )(x, ipsel)


def _pa_body(sel_ref, pwt_ref, pa_ref, ssum_ref, *, k_blocks):
    s = pl.program_id(1)
    k = pl.program_id(2)
    part = lax.dot_general(sel_ref[0], pwt_ref[0], (((1,), (0,)), ((), ())),
                           preferred_element_type=jnp.float32)

    @pl.when(k == 0)
    def _():
        pa_ref[0] = part

    @pl.when(k > 0)
    def _():
        pa_ref[0] = pa_ref[0] + part

    @pl.when(k == k_blocks - 1)
    def _():
        a = _gelu(pa_ref[0])
        pa_ref[0] = a
        colsum = jnp.sum(a, axis=0, keepdims=True)

        @pl.when(s == 0)
        def _():
            ssum_ref[0] = colsum

        @pl.when(s > 0)
        def _():
            ssum_ref[0] = ssum_ref[0] + colsum


def _process_acts(sel_in, pwtsel):
    bs, bk = 512, 512
    k_blocks = _K_IN // bk
    return pl.pallas_call(
        functools.partial(_pa_body, k_blocks=k_blocks),
        grid=(_B, _S // bs, k_blocks),
        in_specs=[
            pl.BlockSpec((1, bs, bk), lambda b, s, k: (b, s, k)),
            pl.BlockSpec((1, bk, _N_PROC), lambda b, s, k: (b, k, 0)),
        ],
        out_specs=[
            pl.BlockSpec((1, bs, _N_PROC), lambda b, s, k: (b, s, 0)),
            pl.BlockSpec((1, 1, _N_PROC), lambda b, s, k: (b, 0, 0)),
        ],
        out_shape=[
            jax.ShapeDtypeStruct((_B, _S, _N_PROC), jnp.float32),
            jax.ShapeDtypeStruct((_B, 1, _N_PROC), jnp.float32),
        ],
    )(sel_in, pwtsel)


def _out_body(pa_ref, po_ref, idx2_ref, out_ref, mask_ref):
    s = pl.program_id(1)

    @pl.when(s == 0)
    def _():
        ids = idx2_ref[0]
        piota = lax.broadcasted_iota(jnp.int32, (_K_PROC, _N_PROC), 1)
        hits = (ids[0][:, None] == piota).astype(jnp.float32)
        mask_ref[...] = jnp.max(hits, axis=0, keepdims=True)

    po_m = po_ref[...] * mask_ref[0][:, None]
    out_ref[0] = lax.dot_general(pa_ref[0], po_m, (((1,), (0,)), ((), ())),
                                 preferred_element_type=jnp.float32)


def _out_mm(pa, po, idx2):
    bs = 512
    return pl.pallas_call(
        _out_body,
        grid=(_B, _S // bs),
        in_specs=[
            pl.BlockSpec((1, bs, _N_PROC), lambda b, s: (b, s, 0)),
            pl.BlockSpec((_N_PROC, _D), lambda b, s: (0, 0)),
            pl.BlockSpec((1, 1, _K_PROC), lambda b, s: (b, 0, 0)),
        ],
        out_specs=pl.BlockSpec((1, bs, _D), lambda b, s: (b, s, 0)),
        out_shape=jax.ShapeDtypeStruct((_B, _S, _D), jnp.float32),
        scratch_shapes=[pltpu.VMEM((1, _N_PROC), jnp.float32)],
    )(pa, po, idx2)


def kernel(x, W1, b1, ln_g, ln_b, W2, b2, neuron_keys, input_patterns,
           process_weights, process_outputs, k_input, k_process):
    logits = _router(x, W1, b1, ln_g, ln_b, W2, b2, neuron_keys)
    _, input_idx = lax.top_k(logits, _K_IN)
    idx1_flat = input_idx.reshape(-1).astype(jnp.int32)

    pwt = process_weights.T
    ipsel = _sc_gather(input_patterns, idx1_flat, _B * _K_IN, _D, 64)
    pwtsel = _sc_gather(pwt, idx1_flat, _B * _K_IN, _N_PROC, 32)
    ipsel = ipsel.reshape(_B, _K_IN, _D)
    pwtsel = pwtsel.reshape(_B, _K_IN, _N_PROC)

    sel_in = _selin(x, ipsel)
    pa, ssum = _process_acts(sel_in, pwtsel)
    _, process_idx = lax.top_k(ssum.reshape(_B, _N_PROC), _K_PROC)
    idx2 = process_idx.astype(jnp.int32).reshape(_B, 1, _K_PROC)

    return _out_mm(pa, process_outputs, idx2)


# speedup vs baseline: 3.4948x; 1.0715x over previous
import functools

import jax
import jax.numpy as jnp
from jax import lax
from jax.experimental import pallas as pl
from jax.experimental.pallas import tpu as pltpu
from jax.experimental.pallas import tpu_sc as plsc

_B, _S, _D = 2, 2048, 768
_N_IN, _N_PROC, _D_R = 4096, 2048, 256
_K_IN, _K_PROC = 2048, 1024
_TEMP = 0.5


def _gelu(v):
    return 0.5 * v * (1.0 + lax.erf(v * (2.0 ** -0.5)))


def _router_body(x_ref, w1_ref, b1_ref, g_ref, bln_ref, w2_ref, b2_ref,
                 nk_ref, logits_ref):
    gc = jnp.max(x_ref[...], axis=1)
    h = lax.dot_general(gc, w1_ref[...], (((1,), (1,)), ((), ())),
                        preferred_element_type=jnp.float32) + b1_ref[...]
    h = _gelu(h)
    mu = jnp.mean(h, axis=-1, keepdims=True)
    var = jnp.mean((h - mu) ** 2, axis=-1, keepdims=True)
    h = (h - mu) / jnp.sqrt(var + 1e-5) * g_ref[...] + bln_ref[...]
    q = lax.dot_general(h, w2_ref[...], (((1,), (1,)), ((), ())),
                        preferred_element_type=jnp.float32) + b2_ref[...]
    logits_ref[...] = lax.dot_general(
        q, nk_ref[...], (((1,), (1,)), ((), ())),
        preferred_element_type=jnp.float32) * (_D_R ** -0.5)


def _router(x, w1, b1, g, bln, w2, b2, nk):
    return pl.pallas_call(
        _router_body,
        out_shape=jax.ShapeDtypeStruct((_B, _N_IN), jnp.float32),
    )(x, w1, b1, g, bln, w2, b2, nk)


def _sc_gather(table, idx_flat, rows_total, d, chunk):
    info = plsc.get_sparse_core_info()
    nc, ns = info.num_cores, info.num_subcores
    nw = nc * ns
    per_w = rows_total // nw
    mesh = plsc.VectorSubcoreMesh(core_axis_name="c", subcore_axis_name="s")

    @functools.partial(
        pl.kernel, mesh=mesh,
        out_type=jax.ShapeDtypeStruct((rows_total, d), jnp.float32),
        scratch_types=[
            pltpu.VMEM((chunk,), jnp.int32),
            pltpu.VMEM((chunk, d), jnp.float32),
            pltpu.SemaphoreType.DMA,
        ],
    )
    def k(table_hbm, idx_hbm, out_hbm, idx_v, rows_v, sem):
        wid = lax.axis_index("s") * nc + lax.axis_index("c")
        base = wid * per_w
        for c in range(per_w // chunk):
            off = base + c * chunk
            pltpu.sync_copy(idx_hbm.at[pl.ds(off, chunk)], idx_v)
            pltpu.async_copy(table_hbm.at[idx_v], rows_v, sem).wait()
            pltpu.sync_copy(rows_v, out_hbm.at[pl.ds(off, chunk)])

    return k(table, idx_flat)


def _selin_body(x_ref, ip_ref, out_ref):
    out_ref[0] = _gelu(lax.dot_general(
        x_ref[0], ip_ref[0].astype(jnp.bfloat16), (((1,), (1,)), ((), ())),
        preferred_element_type=jnp.float32)).astype(jnp.bfloat16)


def _selin(x_bf, ipsel):
    bs = 512
    return pl.pallas_call(
        _selin_body,
        grid=(_B, _S // bs),
        in_specs=[
            pl.BlockSpec((1, bs, _D), lambda b, s: (b, s, 0)),
            pl.BlockSpec((1, _K_IN, _D), lambda b, s: (b, 0, 0)),
        ],
        out_specs=pl.BlockSpec((1, bs, _K_IN), lambda b, s: (b, s, 0)),
        out_shape=jax.ShapeDtypeStruct((_B, _S, _K_IN), jnp.bfloat16),
    )(x_bf, ipsel)


def _pa_body(sel_ref, pwt_ref, pa_ref, ssum_ref, acc_ref, *, k_blocks):
    s = pl.program_id(1)
    k = pl.program_id(2)
    part = lax.dot_general(sel_ref[0], pwt_ref[0].astype(jnp.bfloat16),
                           (((1,), (0,)), ((), ())),
                           preferred_element_type=jnp.float32)

    @pl.when(k == 0)
    def _():
        acc_ref[...] = part

    @pl.when(k > 0)
    def _():
        acc_ref[...] = acc_ref[...] + part

    @pl.when(k == k_blocks - 1)
    def _():
        a = _gelu(acc_ref[...])
        pa_ref[0] = a.astype(jnp.bfloat16)
        colsum = jnp.sum(a, axis=0, keepdims=True)

        @pl.when(s == 0)
        def _():
            ssum_ref[0] = colsum

        @pl.when(s > 0)
        def _():
            ssum_ref[0] = ssum_ref[0] + colsum


def _process_acts(sel_in, pwtsel):
    bs, bk = 512, 512
    k_blocks = _K_IN // bk
    return pl.pallas_call(
        functools.partial(_pa_body, k_blocks=k_blocks),
        grid=(_B, _S // bs, k_blocks),
        in_specs=[
            pl.BlockSpec((1, bs, bk), lambda b, s, k: (b, s, k)),
            pl.BlockSpec((1, bk, _N_PROC), lambda b, s, k: (b, k, 0)),
        ],
        out_specs=[
            pl.BlockSpec((1, bs, _N_PROC), lambda b, s, k: (b, s, 0)),
            pl.BlockSpec((1, 1, _N_PROC), lambda b, s, k: (b, 0, 0)),
        ],
        out_shape=[
            jax.ShapeDtypeStruct((_B, _S, _N_PROC), jnp.bfloat16),
            jax.ShapeDtypeStruct((_B, 1, _N_PROC), jnp.float32),
        ],
        scratch_shapes=[pltpu.VMEM((bs, _N_PROC), jnp.float32)],
    )(sel_in, pwtsel)


def _out_body(pa_ref, po_ref, idx2_ref, out_ref, mask_ref):
    s = pl.program_id(1)

    @pl.when(s == 0)
    def _():
        ids = idx2_ref[0]
        piota = lax.broadcasted_iota(jnp.int32, (_K_PROC, _N_PROC), 1)
        hits = (ids[0][:, None] == piota).astype(jnp.float32)
        mask_ref[...] = jnp.max(hits, axis=0, keepdims=True)

    po_m = (po_ref[...] * mask_ref[0][:, None]).astype(jnp.bfloat16)
    out_ref[0] = lax.dot_general(pa_ref[0], po_m, (((1,), (0,)), ((), ())),
                                 preferred_element_type=jnp.float32)


def _out_mm(pa, po, idx2):
    bs = 512
    return pl.pallas_call(
        _out_body,
        grid=(_B, _S // bs),
        in_specs=[
            pl.BlockSpec((1, bs, _N_PROC), lambda b, s: (b, s, 0)),
            pl.BlockSpec((_N_PROC, _D), lambda b, s: (0, 0)),
            pl.BlockSpec((1, 1, _K_PROC), lambda b, s: (b, 0, 0)),
        ],
        out_specs=pl.BlockSpec((1, bs, _D), lambda b, s: (b, s, 0)),
        out_shape=jax.ShapeDtypeStruct((_B, _S, _D), jnp.float32),
        scratch_shapes=[pltpu.VMEM((1, _N_PROC), jnp.float32)],
    )(pa, po, idx2)


def kernel(x, W1, b1, ln_g, ln_b, W2, b2, neuron_keys, input_patterns,
           process_weights, process_outputs, k_input, k_process):
    logits = _router(x, W1, b1, ln_g, ln_b, W2, b2, neuron_keys)
    _, input_idx = lax.top_k(logits, _K_IN)
    idx1_flat = input_idx.reshape(-1).astype(jnp.int32)

    pwt = process_weights.T
    ipsel = _sc_gather(input_patterns, idx1_flat, _B * _K_IN, _D, 64)
    pwtsel = _sc_gather(pwt, idx1_flat, _B * _K_IN, _N_PROC, 32)
    ipsel = ipsel.reshape(_B, _K_IN, _D)
    pwtsel = pwtsel.reshape(_B, _K_IN, _N_PROC)

    sel_in = _selin(x.astype(jnp.bfloat16), ipsel)
    pa, ssum = _process_acts(sel_in, pwtsel)
    _, process_idx = lax.top_k(ssum.reshape(_B, _N_PROC), _K_PROC)
    idx2 = process_idx.astype(jnp.int32).reshape(_B, 1, _K_PROC)

    return _out_mm(pa, process_outputs, idx2)
